# baseline (speedup 1.0000x reference)
import functools

import jax
import jax.numpy as jnp
from jax import lax
from jax.experimental import pallas as pl
from jax.experimental.pallas import tpu as pltpu
from jax.experimental.pallas import tpu_sc as plsc

C, H, W = 256, 200, 200
N = 300
OUT = 7
BIN = 4
PADB = 56
NPAD = 320

NC, NS = 2, 16
NW = NC * NS
NDMA = 5
DMA_ROWS = 2 * PADB
PER_W = NDMA * DMA_ROWS

CB = 256


HB = 40
NHB = H // HB
HALO = 4


def _slide_max_body(x_ref, halo_ref, rois_ref, o_ref, idx_ref):
    @pl.when(jnp.logical_and(pl.program_id(0) == 0, pl.program_id(1) == 0))
    def _():
        x0 = rois_ref[:, 0:1]
        y0 = rois_ref[:, 1:2]
        b = lax.broadcasted_iota(jnp.int32, (NPAD, PADB), 1)
        b = jnp.minimum(b, OUT * OUT - 1)
        i = b // OUT
        j = b - i * OUT
        idx_ref[...] = (y0 + BIN * i) * W + (x0 + BIN * j)

    rows = jnp.concatenate([x_ref[...], halo_ref[0:3]], axis=0)
    a = jnp.maximum(
        jnp.maximum(rows[:, 0 : W - 3, :], rows[:, 1 : W - 2, :]),
        jnp.maximum(rows[:, 2 : W - 1, :], rows[:, 3:W, :]),
    )
    m = jnp.maximum(
        jnp.maximum(a[0:HB], a[1 : HB + 1]),
        jnp.maximum(a[2 : HB + 2], a[3 : HB + 3]),
    )
    o_ref[:, 0 : W - 3, :] = m


def _slide_max(x_hwc, rois_pad):
    return pl.pallas_call(
        _slide_max_body,
        grid=(C // CB, NHB),
        in_specs=[
            pl.BlockSpec((HB, W, CB), lambda c, h: (h, 0, c)),
            pl.BlockSpec(
                (HALO, W, CB),
                lambda c, h: (jnp.minimum((h + 1) * (HB // HALO), H // HALO - 1), 0, c),
            ),
            pl.BlockSpec((NPAD, 4), lambda c, h: (0, 0)),
        ],
        out_specs=[
            pl.BlockSpec((HB, W, CB), lambda c, h: (h, 0, c)),
            pl.BlockSpec((NPAD, PADB), lambda c, h: (0, 0)),
        ],
        out_shape=[
            jax.ShapeDtypeStruct((H, W, C), jnp.float32),
            jax.ShapeDtypeStruct((NPAD, PADB), jnp.int32),
        ],
    )(x_hwc, x_hwc, rois_pad)


def _sc_gather_body(table_hbm, idx_hbm, out_hbm, idx_v, rows_v, gsem, ssem):
    wid = lax.axis_index("s") * NC + lax.axis_index("c")
    pltpu.sync_copy(idx_hbm.at[wid], idx_v)
    gets = [None] * NDMA
    puts = [None] * NDMA
    for k in range(NDMA):
        if k >= 3:
            puts[k - 3].wait()
        gets[k] = pltpu.async_copy(
            table_hbm.at[idx_v.at[k]], rows_v.at[k % 3], gsem
        )
        if k > 0:
            gets[k - 1].wait()
            puts[k - 1] = pltpu.async_copy(
                rows_v.at[(k - 1) % 3],
                out_hbm.at[wid, pl.ds((k - 1) * DMA_ROWS, DMA_ROWS)],
                ssem,
            )
    gets[NDMA - 1].wait()
    puts[NDMA - 1] = pltpu.async_copy(
        rows_v.at[(NDMA - 1) % 3],
        out_hbm.at[wid, pl.ds((NDMA - 1) * DMA_ROWS, DMA_ROWS)],
        ssem,
    )
    for k in range(max(0, NDMA - 3), NDMA):
        puts[k].wait()


def _sc_gather(table, idx):
    mesh = plsc.VectorSubcoreMesh(core_axis_name="c", subcore_axis_name="s")
    f = functools.partial(
        pl.kernel,
        mesh=mesh,
        out_type=jax.ShapeDtypeStruct((NW, PER_W, C), jnp.float32),
        scratch_types=[
            pltpu.VMEM((NDMA, DMA_ROWS), jnp.int32),
            pltpu.VMEM((3, DMA_ROWS, C), jnp.float32),
            pltpu.SemaphoreType.DMA,
            pltpu.SemaphoreType.DMA,
        ],
    )(_sc_gather_body)
    return f(table, idx)


def kernel(x, rois):
    x_hwc = jnp.transpose(x[0], (1, 2, 0))
    rois_pad = jnp.pad(rois[0].astype(jnp.int32), ((0, NPAD - N), (0, 0)))
    table, idx = _slide_max(x_hwc, rois_pad)
    table = table.reshape(H * W, C)
    idx = idx.reshape(NW, NDMA, DMA_ROWS)
    rows = _sc_gather(table, idx)
    out = rows.reshape(NPAD, PADB, C)[:N, : OUT * OUT]
    return out.reshape(N, OUT, OUT, C).transpose(0, 3, 1, 2)


# speedup vs baseline: 1.0024x; 1.0024x over previous
import functools

import jax
import jax.numpy as jnp
from jax import lax
from jax.experimental import pallas as pl
from jax.experimental.pallas import tpu as pltpu
from jax.experimental.pallas import tpu_sc as plsc

C, H, W = 256, 200, 200
N = 300
OUT = 7
BIN = 4
PADB = 56
NPAD = 320

NC, NS = 2, 16
NW = NC * NS
NDMA = 5
DMA_ROWS = 2 * PADB
PER_W = NDMA * DMA_ROWS

CB = 128


HB = 100
NHB = H // HB
HALO = 4


def _slide_max_body(x_ref, halo_ref, rois_ref, o_ref, idx_ref):
    @pl.when(jnp.logical_and(pl.program_id(0) == 0, pl.program_id(1) == 0))
    def _():
        x0 = rois_ref[:, 0:1]
        y0 = rois_ref[:, 1:2]
        b = lax.broadcasted_iota(jnp.int32, (NPAD, PADB), 1)
        b = jnp.minimum(b, OUT * OUT - 1)
        i = b // OUT
        j = b - i * OUT
        idx_ref[...] = (y0 + BIN * i) * W + (x0 + BIN * j)

    rows = jnp.concatenate([x_ref[...], halo_ref[0:3]], axis=0)
    a = jnp.maximum(
        jnp.maximum(rows[:, 0 : W - 3, :], rows[:, 1 : W - 2, :]),
        jnp.maximum(rows[:, 2 : W - 1, :], rows[:, 3:W, :]),
    )
    m = jnp.maximum(
        jnp.maximum(a[0:HB], a[1 : HB + 1]),
        jnp.maximum(a[2 : HB + 2], a[3 : HB + 3]),
    )
    o_ref[:, 0 : W - 3, :] = m


def _slide_max(x_hwc, rois_pad):
    return pl.pallas_call(
        _slide_max_body,
        grid=(C // CB, NHB),
        in_specs=[
            pl.BlockSpec((HB, W, CB), lambda c, h: (h, 0, c)),
            pl.BlockSpec(
                (HALO, W, CB),
                lambda c, h: (jnp.minimum((h + 1) * (HB // HALO), H // HALO - 1), 0, c),
            ),
            pl.BlockSpec((NPAD, 4), lambda c, h: (0, 0)),
        ],
        out_specs=[
            pl.BlockSpec((HB, W, CB), lambda c, h: (h, 0, c)),
            pl.BlockSpec((NPAD, PADB), lambda c, h: (0, 0)),
        ],
        out_shape=[
            jax.ShapeDtypeStruct((H, W, C), jnp.float32),
            jax.ShapeDtypeStruct((NPAD, PADB), jnp.int32),
        ],
    )(x_hwc, x_hwc, rois_pad)


def _sc_gather_body(table_hbm, idx_hbm, out_hbm, idx_v, rows_v, gsem, ssem):
    wid = lax.axis_index("s") * NC + lax.axis_index("c")
    pltpu.sync_copy(idx_hbm.at[wid], idx_v)
    gets = [None] * NDMA
    puts = [None] * NDMA
    for k in range(NDMA):
        if k >= 3:
            puts[k - 3].wait()
        gets[k] = pltpu.async_copy(
            table_hbm.at[idx_v.at[k]], rows_v.at[k % 3], gsem
        )
        if k > 0:
            gets[k - 1].wait()
            puts[k - 1] = pltpu.async_copy(
                rows_v.at[(k - 1) % 3],
                out_hbm.at[wid, pl.ds((k - 1) * DMA_ROWS, DMA_ROWS)],
                ssem,
            )
    gets[NDMA - 1].wait()
    puts[NDMA - 1] = pltpu.async_copy(
        rows_v.at[(NDMA - 1) % 3],
        out_hbm.at[wid, pl.ds((NDMA - 1) * DMA_ROWS, DMA_ROWS)],
        ssem,
    )
    for k in range(max(0, NDMA - 3), NDMA):
        puts[k].wait()


def _sc_gather(table, idx):
    mesh = plsc.VectorSubcoreMesh(core_axis_name="c", subcore_axis_name="s")
    f = functools.partial(
        pl.kernel,
        mesh=mesh,
        out_type=jax.ShapeDtypeStruct((NW, PER_W, C), jnp.float32),
        scratch_types=[
            pltpu.VMEM((NDMA, DMA_ROWS), jnp.int32),
            pltpu.VMEM((3, DMA_ROWS, C), jnp.float32),
            pltpu.SemaphoreType.DMA,
            pltpu.SemaphoreType.DMA,
        ],
    )(_sc_gather_body)
    return f(table, idx)


def kernel(x, rois):
    x_hwc = jnp.transpose(x[0], (1, 2, 0))
    rois_pad = jnp.pad(rois[0].astype(jnp.int32), ((0, NPAD - N), (0, 0)))
    table, idx = _slide_max(x_hwc, rois_pad)
    table = table.reshape(H * W, C)
    idx = idx.reshape(NW, NDMA, DMA_ROWS)
    rows = _sc_gather(table, idx)
    out = rows.reshape(NPAD, PADB, C)[:N, : OUT * OUT]
    return out.reshape(N, OUT, OUT, C).transpose(0, 3, 1, 2)
